# baseline (speedup 1.0000x reference)
import functools

import jax
import jax.numpy as jnp
from jax import lax
from jax.experimental import pallas as pl
from jax.experimental.pallas import tpu as pltpu
from jax.experimental.pallas import tpu_sc as plsc

N_D = 5000
N_M = 5000
N = N_D + N_M
E = 320000
EMB = 128

NC = 2
NS = 16
NW = NC * NS
NP = 10112
ROWS_PER_TILE = NP // NS

CHUNK_S = 48
NCHUNK_S = 212
EPP_S = NCHUNK_S * CHUNK_S
SLOTS = 2
RBUF = 2 * SLOTS

CHUNK_D = 88
NCHUNK_D = 116
EPP_D = NCHUNK_D * CHUNK_D
DBUF = 4

_MESH = plsc.VectorSubcoreMesh(core_axis_name="c", subcore_axis_name="s")


def _worker_ids():
  c = lax.axis_index("c")
  s = lax.axis_index("s")
  return c, s, s * NC + c


def _seg_sum(h, src3, dst3, z2):
  def body(h_hbm, src_hbm, dst_hbm, z2_hbm, agg_hbm, *rest):
    rows = rest[0:RBUF]
    o = RBUF
    idx_s = [rest[o + 2 * b: o + 2 * b + 2] for b in range(SLOTS)]
    o += 2 * SLOTS
    idx_d = [rest[o + 2 * b: o + 2 * b + 2] for b in range(SLOTS)]
    o += 2 * SLOTS
    acc_sh = rest[o]
    o += 1
    gsem = rest[o: o + RBUF]
    o += RBUF
    ssem = rest[o: o + RBUF]
    o += RBUF
    isem = [rest[o + 2 * b: o + 2 * b + 2] for b in range(SLOTS)]

    c, s, wid = _worker_ids()
    r0 = s * ROWS_PER_TILE

    pltpu.sync_copy(z2_hbm.at[pl.ds(r0, ROWS_PER_TILE)],
                    acc_sh.at[pl.ds(r0, ROWS_PER_TILE)])

    for b in range(SLOTS):
      for gen in range(2):
        jj = gen * SLOTS + b
        pltpu.async_copy(src_hbm.at[wid, jj], idx_s[b][gen], isem[b][gen])
        pltpu.async_copy(dst_hbm.at[wid, jj], idx_d[b][gen], isem[b][gen])
    for b in range(SLOTS):
      pltpu.make_async_copy(src_hbm.at[wid, b], idx_s[b][0],
                            isem[b][0]).wait()
      pltpu.make_async_copy(dst_hbm.at[wid, b], idx_d[b][0],
                            isem[b][0]).wait()
      pltpu.async_copy(h_hbm.at[idx_s[b][0]], rows[b], gsem[b])

    plsc.subcore_barrier()

    def pair_body(g2, carry):
      for p in range(2):
        g = g2 * 2 + p
        for b in range(SLOTS):
          j = g * SLOTS + b
          r = p * SLOTS + b
          r2 = (1 - p) * SLOTS + b
          pltpu.make_async_copy(h_hbm.at[idx_s[b][p]], rows[r],
                                gsem[r]).wait()
          pltpu.async_copy(rows[r], acc_sh.at[idx_d[b][p]], ssem[r],
                           add=True)

          @pl.when(j + 2 * SLOTS < NCHUNK_S)
          def _():
            pltpu.async_copy(src_hbm.at[wid, j + 2 * SLOTS], idx_s[b][p],
                             isem[b][p])
            pltpu.async_copy(dst_hbm.at[wid, j + 2 * SLOTS], idx_d[b][p],
                             isem[b][p])

          @pl.when(j + SLOTS < NCHUNK_S)
          def _():
            pltpu.make_async_copy(src_hbm.at[wid, j + SLOTS],
                                  idx_s[b][1 - p], isem[b][1 - p]).wait()
            pltpu.make_async_copy(dst_hbm.at[wid, j + SLOTS],
                                  idx_d[b][1 - p], isem[b][1 - p]).wait()

          @pl.when((j + SLOTS < NCHUNK_S) & (j >= SLOTS))
          def _():
            pltpu.make_async_copy(rows[r2], acc_sh.at[idx_d[b][1 - p]],
                                  ssem[r2]).wait()

          @pl.when(j + SLOTS < NCHUNK_S)
          def _():
            pltpu.async_copy(h_hbm.at[idx_s[b][1 - p]], rows[r2], gsem[r2])
      return carry

    lax.fori_loop(0, NCHUNK_S // (2 * SLOTS), pair_body, 0)

    for r in range(RBUF):
      pltpu.make_async_copy(rows[r], acc_sh.at[idx_d[r % SLOTS][0]],
                            ssem[r]).wait()
    plsc.subcore_barrier()

    pltpu.sync_copy(acc_sh.at[pl.ds(r0, ROWS_PER_TILE)],
                    agg_hbm.at[c, pl.ds(r0, ROWS_PER_TILE)])

  return pl.kernel(
      body,
      out_type=jax.ShapeDtypeStruct((NC, NP, EMB), jnp.float32),
      mesh=_MESH,
      scratch_types=(
          [pltpu.VMEM((CHUNK_S, EMB), jnp.float32) for _ in range(RBUF)]
          + [pltpu.VMEM((CHUNK_S,), jnp.int32) for _ in range(2 * SLOTS)]
          + [pltpu.VMEM((CHUNK_S,), jnp.int32) for _ in range(2 * SLOTS)]
          + [pltpu.VMEM_SHARED((NP, EMB), jnp.float32)]
          + [pltpu.SemaphoreType.DMA for _ in range(2 * RBUF)]
          + [pltpu.SemaphoreType.DMA for _ in range(2 * SLOTS)]
      ),
  )(h, src3, dst3, z2)


def _deg_count(dst3, z2, ones):
  def body(dst_hbm, z2_hbm, ones_hbm, deg_hbm, *rest):
    idx_d = rest[0]
    ones_v = rest[1]
    acc_sh = rest[2]
    ssem = rest[3:3 + DBUF]
    c, s, wid = _worker_ids()
    r0 = s * ROWS_PER_TILE

    pltpu.sync_copy(z2_hbm.at[pl.ds(r0, ROWS_PER_TILE)],
                    acc_sh.at[pl.ds(r0, ROWS_PER_TILE)])
    pltpu.sync_copy(ones_hbm, ones_v)
    pltpu.sync_copy(dst_hbm.at[wid], idx_d)
    plsc.subcore_barrier()

    for b in range(DBUF):
      pltpu.async_copy(ones_v, acc_sh.at[idx_d.at[b]], ssem[b], add=True)

    def group_body(g, carry):
      for b in range(DBUF):
        j = g * DBUF + b
        pltpu.make_async_copy(ones_v, acc_sh.at[idx_d.at[j]], ssem[b]).wait()

        @pl.when(j + DBUF < NCHUNK_D)
        def _():
          pltpu.async_copy(ones_v, acc_sh.at[idx_d.at[j + DBUF]], ssem[b],
                           add=True)
      return carry

    lax.fori_loop(0, NCHUNK_D // DBUF, group_body, 0)
    plsc.subcore_barrier()

    pltpu.sync_copy(acc_sh.at[pl.ds(r0, ROWS_PER_TILE)],
                    deg_hbm.at[c, pl.ds(r0, ROWS_PER_TILE)])

  return pl.kernel(
      body,
      out_type=jax.ShapeDtypeStruct((NC, NP, EMB), jnp.float32),
      mesh=_MESH,
      scratch_types=(
          [pltpu.VMEM((NCHUNK_D, CHUNK_D), jnp.int32),
           pltpu.VMEM((CHUNK_D, EMB), jnp.float32),
           pltpu.VMEM_SHARED((NP, EMB), jnp.float32)]
          + [pltpu.SemaphoreType.DMA for _ in range(DBUF)]
      ),
  )(dst3, z2, ones)


def _linear(x, w, b, block_rows):
  m, k = x.shape
  _, o = w.shape

  def body(x_ref, w_ref, b_ref, o_ref):
    o_ref[...] = (
        jnp.dot(x_ref[...], w_ref[...], preferred_element_type=jnp.float32)
        + b_ref[...])

  return pl.pallas_call(
      body,
      grid=(m // block_rows,),
      in_specs=[
          pl.BlockSpec((block_rows, k), lambda i: (i, 0)),
          pl.BlockSpec((k, o), lambda i: (0, 0)),
          pl.BlockSpec((o,), lambda i: (0,)),
      ],
      out_specs=pl.BlockSpec((block_rows, o), lambda i: (i, 0)),
      out_shape=jax.ShapeDtypeStruct((m, o), jnp.float32),
  )(x, w, b)


def _sage_layer(hx, pp, degs, w_s, w_n, b, relu, block_rows=1264):
  m, k = hx.shape
  _, o = w_s.shape

  def body(h_ref, pp_ref, d_ref, ws_ref, wn_ref, b_ref, o_ref):
    recip = 1.0 / jnp.maximum(d_ref[0] + d_ref[1], 1.0)
    agg = (pp_ref[0] + pp_ref[1]) * recip
    acc = jnp.dot(h_ref[...], ws_ref[...], preferred_element_type=jnp.float32)
    acc = acc + jnp.dot(agg, wn_ref[...], preferred_element_type=jnp.float32)
    acc = acc + b_ref[...]
    if relu:
      acc = jnp.maximum(acc, 0.0)
    o_ref[...] = acc

  return pl.pallas_call(
      body,
      grid=(m // block_rows,),
      in_specs=[
          pl.BlockSpec((block_rows, k), lambda i: (i, 0)),
          pl.BlockSpec((NC, block_rows, k), lambda i: (0, i, 0)),
          pl.BlockSpec((NC, block_rows, 1), lambda i: (0, i, 0)),
          pl.BlockSpec((k, o), lambda i: (0, 0)),
          pl.BlockSpec((k, o), lambda i: (0, 0)),
          pl.BlockSpec((o,), lambda i: (0,)),
      ],
      out_specs=pl.BlockSpec((block_rows, o), lambda i: (i, 0)),
      out_shape=jax.ShapeDtypeStruct((m, o), jnp.float32),
  )(hx, pp, degs, w_s, w_n, b)


def kernel(d_features, m_features, edge_index, W_d, b_d, W_m, b_m,
           W_self1, W_neigh1, b1, W_self2, W_neigh2, b2):
  ei = edge_index.astype(jnp.int32)
  npad_s = NW * EPP_S - E
  pad_s = jnp.arange(npad_s, dtype=jnp.int32)
  src3 = jnp.concatenate([ei[0], pad_s % N]).reshape(NW, NCHUNK_S, CHUNK_S)
  dst3 = jnp.concatenate([ei[1], N + pad_s % (NP - N)]
                         ).reshape(NW, NCHUNK_S, CHUNK_S)
  npad_d = NW * EPP_D - E
  pad_d = jnp.arange(npad_d, dtype=jnp.int32)
  dst3d = jnp.concatenate([ei[1], N + pad_d % (NP - N)]
                          ).reshape(NW, NCHUNK_D, CHUNK_D)

  z2 = jnp.zeros((NP, EMB), jnp.float32)
  ones = jnp.ones((CHUNK_D, EMB), jnp.float32)

  degp = _deg_count(dst3d, z2, ones)
  degs = degp[:, :, 0:1]

  h_d = _linear(d_features, W_d, b_d, 1000)
  h_m = _linear(m_features, W_m, b_m, 1000)
  h = jnp.concatenate([h_d, h_m, jnp.zeros((NP - N, EMB), jnp.float32)],
                      axis=0)

  agg1 = _seg_sum(h, src3, dst3, z2)
  h1 = _sage_layer(h, agg1, degs, W_self1, W_neigh1, b1, True)

  agg2 = _seg_sum(h1, src3, dst3, z2)
  h2 = _sage_layer(h1, agg2, degs, W_self2, W_neigh2, b2, False)
  return h2[:N]


# speedup vs baseline: 1.0837x; 1.0837x over previous
import functools

import jax
import jax.numpy as jnp
from jax import lax
from jax.experimental import pallas as pl
from jax.experimental.pallas import tpu as pltpu
from jax.experimental.pallas import tpu_sc as plsc

N_D = 5000
N_M = 5000
N = N_D + N_M
E = 320000
EMB = 128

NC = 2
NS = 16
NW = NC * NS
NP = 10112
ROWS_PER_TILE = NP // NS

CHUNK_S = 88
NCHUNK_S = 116
EPP_S = NCHUNK_S * CHUNK_S
SLOTS = 2

CHUNK_D = 88
NCHUNK_D = 116
EPP_D = NCHUNK_D * CHUNK_D
DBUF = 4

_MESH = plsc.VectorSubcoreMesh(core_axis_name="c", subcore_axis_name="s")


def _worker_ids():
  c = lax.axis_index("c")
  s = lax.axis_index("s")
  return c, s, s * NC + c


def _seg_sum(h, src3, dst3, z2):
  def body(h_hbm, src_hbm, dst_hbm, z2_hbm, agg_hbm, *rest):
    rows = rest[0:SLOTS]
    o = SLOTS
    idx_s = [rest[o + 2 * b: o + 2 * b + 2] for b in range(SLOTS)]
    o += 2 * SLOTS
    idx_d = [rest[o + 2 * b: o + 2 * b + 2] for b in range(SLOTS)]
    o += 2 * SLOTS
    acc_sh = rest[o]
    o += 1
    gsem = rest[o: o + SLOTS]
    o += SLOTS
    isem = [rest[o + 2 * b: o + 2 * b + 2] for b in range(SLOTS)]

    c, s, wid = _worker_ids()
    r0 = s * ROWS_PER_TILE

    pltpu.sync_copy(z2_hbm.at[pl.ds(r0, ROWS_PER_TILE)],
                    acc_sh.at[pl.ds(r0, ROWS_PER_TILE)])

    for b in range(SLOTS):
      for gen in range(2):
        jj = gen * SLOTS + b
        pltpu.async_copy(src_hbm.at[wid, jj], idx_s[b][gen], isem[b][gen])
        pltpu.async_copy(dst_hbm.at[wid, jj], idx_d[b][gen], isem[b][gen])
    for b in range(SLOTS):
      pltpu.make_async_copy(src_hbm.at[wid, b], idx_s[b][0],
                            isem[b][0]).wait()
      pltpu.make_async_copy(dst_hbm.at[wid, b], idx_d[b][0],
                            isem[b][0]).wait()
      pltpu.async_copy(h_hbm.at[idx_s[b][0]], rows[b], gsem[b])

    plsc.subcore_barrier()

    def pair_body(g2, carry):
      for p in range(2):
        g = g2 * 2 + p
        for b in range(SLOTS):
          j = g * SLOTS + b
          pltpu.make_async_copy(h_hbm.at[idx_s[b][p]], rows[b],
                                gsem[b]).wait()
          pltpu.sync_copy(rows[b], acc_sh.at[idx_d[b][p]], add=True)

          @pl.when(j + 2 * SLOTS < NCHUNK_S)
          def _():
            pltpu.async_copy(src_hbm.at[wid, j + 2 * SLOTS], idx_s[b][p],
                             isem[b][p])
            pltpu.async_copy(dst_hbm.at[wid, j + 2 * SLOTS], idx_d[b][p],
                             isem[b][p])

          @pl.when(j + SLOTS < NCHUNK_S)
          def _():
            pltpu.make_async_copy(src_hbm.at[wid, j + SLOTS],
                                  idx_s[b][1 - p], isem[b][1 - p]).wait()
            pltpu.make_async_copy(dst_hbm.at[wid, j + SLOTS],
                                  idx_d[b][1 - p], isem[b][1 - p]).wait()
            pltpu.async_copy(h_hbm.at[idx_s[b][1 - p]], rows[b], gsem[b])
      return carry

    lax.fori_loop(0, NCHUNK_S // (2 * SLOTS), pair_body, 0)
    plsc.subcore_barrier()

    pltpu.sync_copy(acc_sh.at[pl.ds(r0, ROWS_PER_TILE)],
                    agg_hbm.at[c, pl.ds(r0, ROWS_PER_TILE)])

  return pl.kernel(
      body,
      out_type=jax.ShapeDtypeStruct((NC, NP, EMB), jnp.float32),
      mesh=_MESH,
      scratch_types=(
          [pltpu.VMEM((CHUNK_S, EMB), jnp.float32) for _ in range(SLOTS)]
          + [pltpu.VMEM((CHUNK_S,), jnp.int32) for _ in range(2 * SLOTS)]
          + [pltpu.VMEM((CHUNK_S,), jnp.int32) for _ in range(2 * SLOTS)]
          + [pltpu.VMEM_SHARED((NP, EMB), jnp.float32)]
          + [pltpu.SemaphoreType.DMA for _ in range(SLOTS)]
          + [pltpu.SemaphoreType.DMA for _ in range(2 * SLOTS)]
      ),
  )(h, src3, dst3, z2)


def _deg_count(dst3, z2, ones):
  def body(dst_hbm, z2_hbm, ones_hbm, deg_hbm, *rest):
    idx_d = rest[0]
    ones_v = rest[1]
    acc_sh = rest[2]
    ssem = rest[3:3 + DBUF]
    c, s, wid = _worker_ids()
    r0 = s * ROWS_PER_TILE

    pltpu.sync_copy(z2_hbm.at[pl.ds(r0, ROWS_PER_TILE)],
                    acc_sh.at[pl.ds(r0, ROWS_PER_TILE)])
    pltpu.sync_copy(ones_hbm, ones_v)
    pltpu.sync_copy(dst_hbm.at[wid], idx_d)
    plsc.subcore_barrier()

    for b in range(DBUF):
      pltpu.async_copy(ones_v, acc_sh.at[idx_d.at[b]], ssem[b], add=True)

    def group_body(g, carry):
      for b in range(DBUF):
        j = g * DBUF + b
        pltpu.make_async_copy(ones_v, acc_sh.at[idx_d.at[j]], ssem[b]).wait()

        @pl.when(j + DBUF < NCHUNK_D)
        def _():
          pltpu.async_copy(ones_v, acc_sh.at[idx_d.at[j + DBUF]], ssem[b],
                           add=True)
      return carry

    lax.fori_loop(0, NCHUNK_D // DBUF, group_body, 0)
    plsc.subcore_barrier()

    pltpu.sync_copy(acc_sh.at[pl.ds(r0, ROWS_PER_TILE)],
                    deg_hbm.at[c, pl.ds(r0, ROWS_PER_TILE)])

  return pl.kernel(
      body,
      out_type=jax.ShapeDtypeStruct((NC, NP, EMB), jnp.float32),
      mesh=_MESH,
      scratch_types=(
          [pltpu.VMEM((NCHUNK_D, CHUNK_D), jnp.int32),
           pltpu.VMEM((CHUNK_D, EMB), jnp.float32),
           pltpu.VMEM_SHARED((NP, EMB), jnp.float32)]
          + [pltpu.SemaphoreType.DMA for _ in range(DBUF)]
      ),
  )(dst3, z2, ones)


def _linear(x, w, b, block_rows):
  m, k = x.shape
  _, o = w.shape

  def body(x_ref, w_ref, b_ref, o_ref):
    o_ref[...] = (
        jnp.dot(x_ref[...], w_ref[...], preferred_element_type=jnp.float32)
        + b_ref[...])

  return pl.pallas_call(
      body,
      grid=(m // block_rows,),
      in_specs=[
          pl.BlockSpec((block_rows, k), lambda i: (i, 0)),
          pl.BlockSpec((k, o), lambda i: (0, 0)),
          pl.BlockSpec((o,), lambda i: (0,)),
      ],
      out_specs=pl.BlockSpec((block_rows, o), lambda i: (i, 0)),
      out_shape=jax.ShapeDtypeStruct((m, o), jnp.float32),
  )(x, w, b)


def _sage_layer(hx, pp, degs, w_s, w_n, b, relu, block_rows=1264):
  m, k = hx.shape
  _, o = w_s.shape

  def body(h_ref, pp_ref, d_ref, ws_ref, wn_ref, b_ref, o_ref):
    recip = 1.0 / jnp.maximum(d_ref[0] + d_ref[1], 1.0)
    agg = (pp_ref[0] + pp_ref[1]) * recip
    acc = jnp.dot(h_ref[...], ws_ref[...], preferred_element_type=jnp.float32)
    acc = acc + jnp.dot(agg, wn_ref[...], preferred_element_type=jnp.float32)
    acc = acc + b_ref[...]
    if relu:
      acc = jnp.maximum(acc, 0.0)
    o_ref[...] = acc

  return pl.pallas_call(
      body,
      grid=(m // block_rows,),
      in_specs=[
          pl.BlockSpec((block_rows, k), lambda i: (i, 0)),
          pl.BlockSpec((NC, block_rows, k), lambda i: (0, i, 0)),
          pl.BlockSpec((NC, block_rows, 1), lambda i: (0, i, 0)),
          pl.BlockSpec((k, o), lambda i: (0, 0)),
          pl.BlockSpec((k, o), lambda i: (0, 0)),
          pl.BlockSpec((o,), lambda i: (0,)),
      ],
      out_specs=pl.BlockSpec((block_rows, o), lambda i: (i, 0)),
      out_shape=jax.ShapeDtypeStruct((m, o), jnp.float32),
  )(hx, pp, degs, w_s, w_n, b)


def kernel(d_features, m_features, edge_index, W_d, b_d, W_m, b_m,
           W_self1, W_neigh1, b1, W_self2, W_neigh2, b2):
  ei = edge_index.astype(jnp.int32)
  npad_s = NW * EPP_S - E
  pad_s = jnp.arange(npad_s, dtype=jnp.int32)
  src3 = jnp.concatenate([ei[0], pad_s % N]).reshape(NW, NCHUNK_S, CHUNK_S)
  dst3 = jnp.concatenate([ei[1], N + pad_s % (NP - N)]
                         ).reshape(NW, NCHUNK_S, CHUNK_S)
  npad_d = NW * EPP_D - E
  pad_d = jnp.arange(npad_d, dtype=jnp.int32)
  dst3d = jnp.concatenate([ei[1], N + pad_d % (NP - N)]
                          ).reshape(NW, NCHUNK_D, CHUNK_D)

  z2 = jnp.zeros((NP, EMB), jnp.float32)
  ones = jnp.ones((CHUNK_D, EMB), jnp.float32)

  degp = _deg_count(dst3d, z2, ones)
  degs = degp[:, :, 0:1]

  h_d = _linear(d_features, W_d, b_d, 1000)
  h_m = _linear(m_features, W_m, b_m, 1000)
  h = jnp.concatenate([h_d, h_m, jnp.zeros((NP - N, EMB), jnp.float32)],
                      axis=0)

  agg1 = _seg_sum(h, src3, dst3, z2)
  h1 = _sage_layer(h, agg1, degs, W_self1, W_neigh1, b1, True)

  agg2 = _seg_sum(h1, src3, dst3, z2)
  h2 = _sage_layer(h1, agg2, degs, W_self2, W_neigh2, b2, False)
  return h2[:N]


# speedup vs baseline: 1.1408x; 1.0527x over previous
import functools

import jax
import jax.numpy as jnp
from jax import lax
from jax.experimental import pallas as pl
from jax.experimental.pallas import tpu as pltpu
from jax.experimental.pallas import tpu_sc as plsc

N_D = 5000
N_M = 5000
N = N_D + N_M
E = 320000
EMB = 128

NC = 2
NS = 16
NW = NC * NS
NP = 10112
ROWS_PER_TILE = NP // NS

CHUNK_S = 56
NCHUNK_S = 180
EPP_S = NCHUNK_S * CHUNK_S
SLOTS = 3

CHUNK_D = 88
NCHUNK_D = 116
EPP_D = NCHUNK_D * CHUNK_D
DBUF = 4

_MESH = plsc.VectorSubcoreMesh(core_axis_name="c", subcore_axis_name="s")


def _worker_ids():
  c = lax.axis_index("c")
  s = lax.axis_index("s")
  return c, s, s * NC + c


def _seg_sum(h, src3, dst3, z2):
  def body(h_hbm, src_hbm, dst_hbm, z2_hbm, agg_hbm, *rest):
    rows = rest[0:SLOTS]
    o = SLOTS
    idx_s = [rest[o + 2 * b: o + 2 * b + 2] for b in range(SLOTS)]
    o += 2 * SLOTS
    idx_d = [rest[o + 2 * b: o + 2 * b + 2] for b in range(SLOTS)]
    o += 2 * SLOTS
    acc_sh = rest[o]
    o += 1
    gsem = rest[o: o + SLOTS]
    o += SLOTS
    isem = [rest[o + 2 * b: o + 2 * b + 2] for b in range(SLOTS)]

    c, s, wid = _worker_ids()
    r0 = s * ROWS_PER_TILE

    pltpu.sync_copy(z2_hbm.at[pl.ds(r0, ROWS_PER_TILE)],
                    acc_sh.at[pl.ds(r0, ROWS_PER_TILE)])

    for b in range(SLOTS):
      for gen in range(2):
        jj = gen * SLOTS + b
        pltpu.async_copy(src_hbm.at[wid, jj], idx_s[b][gen], isem[b][gen])
        pltpu.async_copy(dst_hbm.at[wid, jj], idx_d[b][gen], isem[b][gen])
    for b in range(SLOTS):
      pltpu.make_async_copy(src_hbm.at[wid, b], idx_s[b][0],
                            isem[b][0]).wait()
      pltpu.make_async_copy(dst_hbm.at[wid, b], idx_d[b][0],
                            isem[b][0]).wait()
      pltpu.async_copy(h_hbm.at[idx_s[b][0]], rows[b], gsem[b])

    plsc.subcore_barrier()

    def pair_body(g2, carry):
      for p in range(2):
        g = g2 * 2 + p
        for b in range(SLOTS):
          j = g * SLOTS + b
          pltpu.make_async_copy(h_hbm.at[idx_s[b][p]], rows[b],
                                gsem[b]).wait()
          pltpu.sync_copy(rows[b], acc_sh.at[idx_d[b][p]], add=True)

          @pl.when(j + 2 * SLOTS < NCHUNK_S)
          def _():
            pltpu.async_copy(src_hbm.at[wid, j + 2 * SLOTS], idx_s[b][p],
                             isem[b][p])
            pltpu.async_copy(dst_hbm.at[wid, j + 2 * SLOTS], idx_d[b][p],
                             isem[b][p])

          @pl.when(j + SLOTS < NCHUNK_S)
          def _():
            pltpu.make_async_copy(src_hbm.at[wid, j + SLOTS],
                                  idx_s[b][1 - p], isem[b][1 - p]).wait()
            pltpu.make_async_copy(dst_hbm.at[wid, j + SLOTS],
                                  idx_d[b][1 - p], isem[b][1 - p]).wait()
            pltpu.async_copy(h_hbm.at[idx_s[b][1 - p]], rows[b], gsem[b])
      return carry

    lax.fori_loop(0, NCHUNK_S // (2 * SLOTS), pair_body, 0)
    plsc.subcore_barrier()

    pltpu.sync_copy(acc_sh.at[pl.ds(r0, ROWS_PER_TILE)],
                    agg_hbm.at[c, pl.ds(r0, ROWS_PER_TILE)])

  return pl.kernel(
      body,
      out_type=jax.ShapeDtypeStruct((NC, NP, EMB), jnp.float32),
      mesh=_MESH,
      scratch_types=(
          [pltpu.VMEM((CHUNK_S, EMB), jnp.float32) for _ in range(SLOTS)]
          + [pltpu.VMEM((CHUNK_S,), jnp.int32) for _ in range(2 * SLOTS)]
          + [pltpu.VMEM((CHUNK_S,), jnp.int32) for _ in range(2 * SLOTS)]
          + [pltpu.VMEM_SHARED((NP, EMB), jnp.float32)]
          + [pltpu.SemaphoreType.DMA for _ in range(SLOTS)]
          + [pltpu.SemaphoreType.DMA for _ in range(2 * SLOTS)]
      ),
  )(h, src3, dst3, z2)


def _deg_count(dst3, z2, ones):
  def body(dst_hbm, z2_hbm, ones_hbm, deg_hbm, *rest):
    idx_d = rest[0]
    ones_v = rest[1]
    acc_sh = rest[2]
    ssem = rest[3:3 + DBUF]
    c, s, wid = _worker_ids()
    r0 = s * ROWS_PER_TILE

    pltpu.sync_copy(z2_hbm.at[pl.ds(r0, ROWS_PER_TILE)],
                    acc_sh.at[pl.ds(r0, ROWS_PER_TILE)])
    pltpu.sync_copy(ones_hbm, ones_v)
    pltpu.sync_copy(dst_hbm.at[wid], idx_d)
    plsc.subcore_barrier()

    for b in range(DBUF):
      pltpu.async_copy(ones_v, acc_sh.at[idx_d.at[b]], ssem[b], add=True)

    def group_body(g, carry):
      for b in range(DBUF):
        j = g * DBUF + b
        pltpu.make_async_copy(ones_v, acc_sh.at[idx_d.at[j]], ssem[b]).wait()

        @pl.when(j + DBUF < NCHUNK_D)
        def _():
          pltpu.async_copy(ones_v, acc_sh.at[idx_d.at[j + DBUF]], ssem[b],
                           add=True)
      return carry

    lax.fori_loop(0, NCHUNK_D // DBUF, group_body, 0)
    plsc.subcore_barrier()

    pltpu.sync_copy(acc_sh.at[pl.ds(r0, ROWS_PER_TILE)],
                    deg_hbm.at[c, pl.ds(r0, ROWS_PER_TILE)])

  return pl.kernel(
      body,
      out_type=jax.ShapeDtypeStruct((NC, NP, EMB), jnp.float32),
      mesh=_MESH,
      scratch_types=(
          [pltpu.VMEM((NCHUNK_D, CHUNK_D), jnp.int32),
           pltpu.VMEM((CHUNK_D, EMB), jnp.float32),
           pltpu.VMEM_SHARED((NP, EMB), jnp.float32)]
          + [pltpu.SemaphoreType.DMA for _ in range(DBUF)]
      ),
  )(dst3, z2, ones)


def _linear(x, w, b, block_rows):
  m, k = x.shape
  _, o = w.shape

  def body(x_ref, w_ref, b_ref, o_ref):
    o_ref[...] = (
        jnp.dot(x_ref[...], w_ref[...], preferred_element_type=jnp.float32)
        + b_ref[...])

  return pl.pallas_call(
      body,
      grid=(m // block_rows,),
      in_specs=[
          pl.BlockSpec((block_rows, k), lambda i: (i, 0)),
          pl.BlockSpec((k, o), lambda i: (0, 0)),
          pl.BlockSpec((o,), lambda i: (0,)),
      ],
      out_specs=pl.BlockSpec((block_rows, o), lambda i: (i, 0)),
      out_shape=jax.ShapeDtypeStruct((m, o), jnp.float32),
  )(x, w, b)


def _sage_layer(hx, pp, degs, w_s, w_n, b, relu, block_rows=1264):
  m, k = hx.shape
  _, o = w_s.shape

  def body(h_ref, pp_ref, d_ref, ws_ref, wn_ref, b_ref, o_ref):
    recip = 1.0 / jnp.maximum(d_ref[0] + d_ref[1], 1.0)
    agg = (pp_ref[0] + pp_ref[1]) * recip
    acc = jnp.dot(h_ref[...], ws_ref[...], preferred_element_type=jnp.float32)
    acc = acc + jnp.dot(agg, wn_ref[...], preferred_element_type=jnp.float32)
    acc = acc + b_ref[...]
    if relu:
      acc = jnp.maximum(acc, 0.0)
    o_ref[...] = acc

  return pl.pallas_call(
      body,
      grid=(m // block_rows,),
      in_specs=[
          pl.BlockSpec((block_rows, k), lambda i: (i, 0)),
          pl.BlockSpec((NC, block_rows, k), lambda i: (0, i, 0)),
          pl.BlockSpec((NC, block_rows, 1), lambda i: (0, i, 0)),
          pl.BlockSpec((k, o), lambda i: (0, 0)),
          pl.BlockSpec((k, o), lambda i: (0, 0)),
          pl.BlockSpec((o,), lambda i: (0,)),
      ],
      out_specs=pl.BlockSpec((block_rows, o), lambda i: (i, 0)),
      out_shape=jax.ShapeDtypeStruct((m, o), jnp.float32),
  )(hx, pp, degs, w_s, w_n, b)


def kernel(d_features, m_features, edge_index, W_d, b_d, W_m, b_m,
           W_self1, W_neigh1, b1, W_self2, W_neigh2, b2):
  ei = edge_index.astype(jnp.int32)
  npad_s = NW * EPP_S - E
  pad_s = jnp.arange(npad_s, dtype=jnp.int32)
  src3 = jnp.concatenate([ei[0], pad_s % N]).reshape(NW, NCHUNK_S, CHUNK_S)
  dst3 = jnp.concatenate([ei[1], N + pad_s % (NP - N)]
                         ).reshape(NW, NCHUNK_S, CHUNK_S)
  npad_d = NW * EPP_D - E
  pad_d = jnp.arange(npad_d, dtype=jnp.int32)
  dst3d = jnp.concatenate([ei[1], N + pad_d % (NP - N)]
                          ).reshape(NW, NCHUNK_D, CHUNK_D)

  z2 = jnp.zeros((NP, EMB), jnp.float32)
  ones = jnp.ones((CHUNK_D, EMB), jnp.float32)

  degp = _deg_count(dst3d, z2, ones)
  degs = degp[:, :, 0:1]

  h_d = _linear(d_features, W_d, b_d, 1000)
  h_m = _linear(m_features, W_m, b_m, 1000)
  h = jnp.concatenate([h_d, h_m, jnp.zeros((NP - N, EMB), jnp.float32)],
                      axis=0)

  agg1 = _seg_sum(h, src3, dst3, z2)
  h1 = _sage_layer(h, agg1, degs, W_self1, W_neigh1, b1, True)

  agg2 = _seg_sum(h1, src3, dst3, z2)
  h2 = _sage_layer(h1, agg2, degs, W_self2, W_neigh2, b2, False)
  return h2[:N]


# speedup vs baseline: 1.1591x; 1.0161x over previous
import functools

import jax
import jax.numpy as jnp
from jax import lax
from jax.experimental import pallas as pl
from jax.experimental.pallas import tpu as pltpu
from jax.experimental.pallas import tpu_sc as plsc

N_D = 5000
N_M = 5000
N = N_D + N_M
E = 320000
EMB = 128

NC = 2
NS = 16
NW = NC * NS
NP = 10112
ROWS_PER_TILE = NP // NS

CHUNK_S = 40
NCHUNK_S = 256
EPP_S = NCHUNK_S * CHUNK_S
SLOTS = 4

CHUNK_D = 88
NCHUNK_D = 116
EPP_D = NCHUNK_D * CHUNK_D
DBUF = 4

_MESH = plsc.VectorSubcoreMesh(core_axis_name="c", subcore_axis_name="s")


def _worker_ids():
  c = lax.axis_index("c")
  s = lax.axis_index("s")
  return c, s, s * NC + c


def _seg_sum(h, src3, dst3, z2):
  def body(h_hbm, src_hbm, dst_hbm, z2_hbm, agg_hbm, *rest):
    rows = rest[0:SLOTS]
    o = SLOTS
    idx_s = [rest[o + 2 * b: o + 2 * b + 2] for b in range(SLOTS)]
    o += 2 * SLOTS
    idx_d = [rest[o + 2 * b: o + 2 * b + 2] for b in range(SLOTS)]
    o += 2 * SLOTS
    acc_sh = rest[o]
    o += 1
    gsem = rest[o: o + SLOTS]
    o += SLOTS
    isem = [rest[o + 2 * b: o + 2 * b + 2] for b in range(SLOTS)]

    c, s, wid = _worker_ids()
    r0 = s * ROWS_PER_TILE

    pltpu.sync_copy(z2_hbm.at[pl.ds(r0, ROWS_PER_TILE)],
                    acc_sh.at[pl.ds(r0, ROWS_PER_TILE)])

    for b in range(SLOTS):
      for gen in range(2):
        jj = gen * SLOTS + b
        pltpu.async_copy(src_hbm.at[wid, jj], idx_s[b][gen], isem[b][gen])
        pltpu.async_copy(dst_hbm.at[wid, jj], idx_d[b][gen], isem[b][gen])
    for b in range(SLOTS):
      pltpu.make_async_copy(src_hbm.at[wid, b], idx_s[b][0],
                            isem[b][0]).wait()
      pltpu.make_async_copy(dst_hbm.at[wid, b], idx_d[b][0],
                            isem[b][0]).wait()
      pltpu.async_copy(h_hbm.at[idx_s[b][0]], rows[b], gsem[b])

    plsc.subcore_barrier()

    def pair_body(g2, carry):
      for p in range(2):
        g = g2 * 2 + p
        for b in range(SLOTS):
          j = g * SLOTS + b
          pltpu.make_async_copy(h_hbm.at[idx_s[b][p]], rows[b],
                                gsem[b]).wait()
          pltpu.sync_copy(rows[b], acc_sh.at[idx_d[b][p]], add=True)

          @pl.when(j + 2 * SLOTS < NCHUNK_S)
          def _():
            pltpu.async_copy(src_hbm.at[wid, j + 2 * SLOTS], idx_s[b][p],
                             isem[b][p])
            pltpu.async_copy(dst_hbm.at[wid, j + 2 * SLOTS], idx_d[b][p],
                             isem[b][p])

          @pl.when(j + SLOTS < NCHUNK_S)
          def _():
            pltpu.make_async_copy(src_hbm.at[wid, j + SLOTS],
                                  idx_s[b][1 - p], isem[b][1 - p]).wait()
            pltpu.make_async_copy(dst_hbm.at[wid, j + SLOTS],
                                  idx_d[b][1 - p], isem[b][1 - p]).wait()
            pltpu.async_copy(h_hbm.at[idx_s[b][1 - p]], rows[b], gsem[b])
      return carry

    lax.fori_loop(0, NCHUNK_S // (2 * SLOTS), pair_body, 0)
    plsc.subcore_barrier()

    pltpu.sync_copy(acc_sh.at[pl.ds(r0, ROWS_PER_TILE)],
                    agg_hbm.at[c, pl.ds(r0, ROWS_PER_TILE)])

  return pl.kernel(
      body,
      out_type=jax.ShapeDtypeStruct((NC, NP, EMB), jnp.float32),
      mesh=_MESH,
      scratch_types=(
          [pltpu.VMEM((CHUNK_S, EMB), jnp.float32) for _ in range(SLOTS)]
          + [pltpu.VMEM((CHUNK_S,), jnp.int32) for _ in range(2 * SLOTS)]
          + [pltpu.VMEM((CHUNK_S,), jnp.int32) for _ in range(2 * SLOTS)]
          + [pltpu.VMEM_SHARED((NP, EMB), jnp.float32)]
          + [pltpu.SemaphoreType.DMA for _ in range(SLOTS)]
          + [pltpu.SemaphoreType.DMA for _ in range(2 * SLOTS)]
      ),
  )(h, src3, dst3, z2)


def _deg_count(dst3, z2, ones):
  def body(dst_hbm, z2_hbm, ones_hbm, deg_hbm, *rest):
    idx_d = rest[0]
    ones_v = rest[1]
    acc_sh = rest[2]
    ssem = rest[3:3 + DBUF]
    c, s, wid = _worker_ids()
    r0 = s * ROWS_PER_TILE

    pltpu.sync_copy(z2_hbm.at[pl.ds(r0, ROWS_PER_TILE)],
                    acc_sh.at[pl.ds(r0, ROWS_PER_TILE)])
    pltpu.sync_copy(ones_hbm, ones_v)
    pltpu.sync_copy(dst_hbm.at[wid], idx_d)
    plsc.subcore_barrier()

    for b in range(DBUF):
      pltpu.async_copy(ones_v, acc_sh.at[idx_d.at[b]], ssem[b], add=True)

    def group_body(g, carry):
      for b in range(DBUF):
        j = g * DBUF + b
        pltpu.make_async_copy(ones_v, acc_sh.at[idx_d.at[j]], ssem[b]).wait()

        @pl.when(j + DBUF < NCHUNK_D)
        def _():
          pltpu.async_copy(ones_v, acc_sh.at[idx_d.at[j + DBUF]], ssem[b],
                           add=True)
      return carry

    lax.fori_loop(0, NCHUNK_D // DBUF, group_body, 0)
    plsc.subcore_barrier()

    pltpu.sync_copy(acc_sh.at[pl.ds(r0, ROWS_PER_TILE)],
                    deg_hbm.at[c, pl.ds(r0, ROWS_PER_TILE)])

  return pl.kernel(
      body,
      out_type=jax.ShapeDtypeStruct((NC, NP, EMB), jnp.float32),
      mesh=_MESH,
      scratch_types=(
          [pltpu.VMEM((NCHUNK_D, CHUNK_D), jnp.int32),
           pltpu.VMEM((CHUNK_D, EMB), jnp.float32),
           pltpu.VMEM_SHARED((NP, EMB), jnp.float32)]
          + [pltpu.SemaphoreType.DMA for _ in range(DBUF)]
      ),
  )(dst3, z2, ones)


def _linear(x, w, b, block_rows):
  m, k = x.shape
  _, o = w.shape

  def body(x_ref, w_ref, b_ref, o_ref):
    o_ref[...] = (
        jnp.dot(x_ref[...], w_ref[...], preferred_element_type=jnp.float32)
        + b_ref[...])

  return pl.pallas_call(
      body,
      grid=(m // block_rows,),
      in_specs=[
          pl.BlockSpec((block_rows, k), lambda i: (i, 0)),
          pl.BlockSpec((k, o), lambda i: (0, 0)),
          pl.BlockSpec((o,), lambda i: (0,)),
      ],
      out_specs=pl.BlockSpec((block_rows, o), lambda i: (i, 0)),
      out_shape=jax.ShapeDtypeStruct((m, o), jnp.float32),
  )(x, w, b)


def _sage_layer(hx, pp, degs, w_s, w_n, b, relu, block_rows=1264):
  m, k = hx.shape
  _, o = w_s.shape

  def body(h_ref, pp_ref, d_ref, ws_ref, wn_ref, b_ref, o_ref):
    recip = 1.0 / jnp.maximum(d_ref[0] + d_ref[1], 1.0)
    agg = (pp_ref[0] + pp_ref[1]) * recip
    acc = jnp.dot(h_ref[...], ws_ref[...], preferred_element_type=jnp.float32)
    acc = acc + jnp.dot(agg, wn_ref[...], preferred_element_type=jnp.float32)
    acc = acc + b_ref[...]
    if relu:
      acc = jnp.maximum(acc, 0.0)
    o_ref[...] = acc

  return pl.pallas_call(
      body,
      grid=(m // block_rows,),
      in_specs=[
          pl.BlockSpec((block_rows, k), lambda i: (i, 0)),
          pl.BlockSpec((NC, block_rows, k), lambda i: (0, i, 0)),
          pl.BlockSpec((NC, block_rows, 1), lambda i: (0, i, 0)),
          pl.BlockSpec((k, o), lambda i: (0, 0)),
          pl.BlockSpec((k, o), lambda i: (0, 0)),
          pl.BlockSpec((o,), lambda i: (0,)),
      ],
      out_specs=pl.BlockSpec((block_rows, o), lambda i: (i, 0)),
      out_shape=jax.ShapeDtypeStruct((m, o), jnp.float32),
  )(hx, pp, degs, w_s, w_n, b)


def kernel(d_features, m_features, edge_index, W_d, b_d, W_m, b_m,
           W_self1, W_neigh1, b1, W_self2, W_neigh2, b2):
  ei = edge_index.astype(jnp.int32)
  npad_s = NW * EPP_S - E
  pad_s = jnp.arange(npad_s, dtype=jnp.int32)
  src3 = jnp.concatenate([ei[0], pad_s % N]).reshape(NW, NCHUNK_S, CHUNK_S)
  dst3 = jnp.concatenate([ei[1], N + pad_s % (NP - N)]
                         ).reshape(NW, NCHUNK_S, CHUNK_S)
  npad_d = NW * EPP_D - E
  pad_d = jnp.arange(npad_d, dtype=jnp.int32)
  dst3d = jnp.concatenate([ei[1], N + pad_d % (NP - N)]
                          ).reshape(NW, NCHUNK_D, CHUNK_D)

  z2 = jnp.zeros((NP, EMB), jnp.float32)
  ones = jnp.ones((CHUNK_D, EMB), jnp.float32)

  degp = _deg_count(dst3d, z2, ones)
  degs = degp[:, :, 0:1]

  h_d = _linear(d_features, W_d, b_d, 1000)
  h_m = _linear(m_features, W_m, b_m, 1000)
  h = jnp.concatenate([h_d, h_m, jnp.zeros((NP - N, EMB), jnp.float32)],
                      axis=0)

  agg1 = _seg_sum(h, src3, dst3, z2)
  h1 = _sage_layer(h, agg1, degs, W_self1, W_neigh1, b1, True)

  agg2 = _seg_sum(h1, src3, dst3, z2)
  h2 = _sage_layer(h1, agg2, degs, W_self2, W_neigh2, b2, False)
  return h2[:N]


# speedup vs baseline: 1.1846x; 1.0220x over previous
import functools

import jax
import jax.numpy as jnp
from jax import lax
from jax.experimental import pallas as pl
from jax.experimental.pallas import tpu as pltpu
from jax.experimental.pallas import tpu_sc as plsc

N_D = 5000
N_M = 5000
N = N_D + N_M
E = 320000
EMB = 128

NC = 2
NS = 16
NW = NC * NS
NP = 10112
ROWS_PER_TILE = NP // NS

CHUNK_S = 32
NCHUNK_S = 320
EPP_S = NCHUNK_S * CHUNK_S
SLOTS = 5

CHUNK_D = 88
NCHUNK_D = 116
EPP_D = NCHUNK_D * CHUNK_D
DBUF = 4

_MESH = plsc.VectorSubcoreMesh(core_axis_name="c", subcore_axis_name="s")


def _worker_ids():
  c = lax.axis_index("c")
  s = lax.axis_index("s")
  return c, s, s * NC + c


def _seg_sum(h, src3, dst3, z2):
  def body(h_hbm, src_hbm, dst_hbm, z2_hbm, agg_hbm, *rest):
    rows = rest[0:SLOTS]
    o = SLOTS
    idx_s = [rest[o + 2 * b: o + 2 * b + 2] for b in range(SLOTS)]
    o += 2 * SLOTS
    idx_d = [rest[o + 2 * b: o + 2 * b + 2] for b in range(SLOTS)]
    o += 2 * SLOTS
    acc_sh = rest[o]
    o += 1
    gsem = rest[o: o + SLOTS]
    o += SLOTS
    isem = [rest[o + 2 * b: o + 2 * b + 2] for b in range(SLOTS)]

    c, s, wid = _worker_ids()
    r0 = s * ROWS_PER_TILE

    pltpu.sync_copy(z2_hbm.at[pl.ds(r0, ROWS_PER_TILE)],
                    acc_sh.at[pl.ds(r0, ROWS_PER_TILE)])

    for b in range(SLOTS):
      for gen in range(2):
        jj = gen * SLOTS + b
        pltpu.async_copy(src_hbm.at[wid, jj], idx_s[b][gen], isem[b][gen])
        pltpu.async_copy(dst_hbm.at[wid, jj], idx_d[b][gen], isem[b][gen])
    for b in range(SLOTS):
      pltpu.make_async_copy(src_hbm.at[wid, b], idx_s[b][0],
                            isem[b][0]).wait()
      pltpu.make_async_copy(dst_hbm.at[wid, b], idx_d[b][0],
                            isem[b][0]).wait()
      pltpu.async_copy(h_hbm.at[idx_s[b][0]], rows[b], gsem[b])

    plsc.subcore_barrier()

    def pair_body(g2, carry):
      for p in range(2):
        g = g2 * 2 + p
        for b in range(SLOTS):
          j = g * SLOTS + b
          pltpu.make_async_copy(h_hbm.at[idx_s[b][p]], rows[b],
                                gsem[b]).wait()
          pltpu.sync_copy(rows[b], acc_sh.at[idx_d[b][p]], add=True)

          @pl.when(j + 2 * SLOTS < NCHUNK_S)
          def _():
            pltpu.async_copy(src_hbm.at[wid, j + 2 * SLOTS], idx_s[b][p],
                             isem[b][p])
            pltpu.async_copy(dst_hbm.at[wid, j + 2 * SLOTS], idx_d[b][p],
                             isem[b][p])

          @pl.when(j + SLOTS < NCHUNK_S)
          def _():
            pltpu.make_async_copy(src_hbm.at[wid, j + SLOTS],
                                  idx_s[b][1 - p], isem[b][1 - p]).wait()
            pltpu.make_async_copy(dst_hbm.at[wid, j + SLOTS],
                                  idx_d[b][1 - p], isem[b][1 - p]).wait()
            pltpu.async_copy(h_hbm.at[idx_s[b][1 - p]], rows[b], gsem[b])
      return carry

    lax.fori_loop(0, NCHUNK_S // (2 * SLOTS), pair_body, 0)
    plsc.subcore_barrier()

    pltpu.sync_copy(acc_sh.at[pl.ds(r0, ROWS_PER_TILE)],
                    agg_hbm.at[c, pl.ds(r0, ROWS_PER_TILE)])

  return pl.kernel(
      body,
      out_type=jax.ShapeDtypeStruct((NC, NP, EMB), jnp.float32),
      mesh=_MESH,
      scratch_types=(
          [pltpu.VMEM((CHUNK_S, EMB), jnp.float32) for _ in range(SLOTS)]
          + [pltpu.VMEM((CHUNK_S,), jnp.int32) for _ in range(2 * SLOTS)]
          + [pltpu.VMEM((CHUNK_S,), jnp.int32) for _ in range(2 * SLOTS)]
          + [pltpu.VMEM_SHARED((NP, EMB), jnp.float32)]
          + [pltpu.SemaphoreType.DMA for _ in range(SLOTS)]
          + [pltpu.SemaphoreType.DMA for _ in range(2 * SLOTS)]
      ),
  )(h, src3, dst3, z2)


def _deg_count(dst3, z2, ones):
  def body(dst_hbm, z2_hbm, ones_hbm, deg_hbm, *rest):
    idx_d = rest[0]
    ones_v = rest[1]
    acc_sh = rest[2]
    ssem = rest[3:3 + DBUF]
    c, s, wid = _worker_ids()
    r0 = s * ROWS_PER_TILE

    pltpu.sync_copy(z2_hbm.at[pl.ds(r0, ROWS_PER_TILE)],
                    acc_sh.at[pl.ds(r0, ROWS_PER_TILE)])
    pltpu.sync_copy(ones_hbm, ones_v)
    pltpu.sync_copy(dst_hbm.at[wid], idx_d)
    plsc.subcore_barrier()

    for b in range(DBUF):
      pltpu.async_copy(ones_v, acc_sh.at[idx_d.at[b]], ssem[b], add=True)

    def group_body(g, carry):
      for b in range(DBUF):
        j = g * DBUF + b
        pltpu.make_async_copy(ones_v, acc_sh.at[idx_d.at[j]], ssem[b]).wait()

        @pl.when(j + DBUF < NCHUNK_D)
        def _():
          pltpu.async_copy(ones_v, acc_sh.at[idx_d.at[j + DBUF]], ssem[b],
                           add=True)
      return carry

    lax.fori_loop(0, NCHUNK_D // DBUF, group_body, 0)
    plsc.subcore_barrier()

    pltpu.sync_copy(acc_sh.at[pl.ds(r0, ROWS_PER_TILE)],
                    deg_hbm.at[c, pl.ds(r0, ROWS_PER_TILE)])

  return pl.kernel(
      body,
      out_type=jax.ShapeDtypeStruct((NC, NP, EMB), jnp.float32),
      mesh=_MESH,
      scratch_types=(
          [pltpu.VMEM((NCHUNK_D, CHUNK_D), jnp.int32),
           pltpu.VMEM((CHUNK_D, EMB), jnp.float32),
           pltpu.VMEM_SHARED((NP, EMB), jnp.float32)]
          + [pltpu.SemaphoreType.DMA for _ in range(DBUF)]
      ),
  )(dst3, z2, ones)


def _linear(x, w, b, block_rows):
  m, k = x.shape
  _, o = w.shape

  def body(x_ref, w_ref, b_ref, o_ref):
    o_ref[...] = (
        jnp.dot(x_ref[...], w_ref[...], preferred_element_type=jnp.float32)
        + b_ref[...])

  return pl.pallas_call(
      body,
      grid=(m // block_rows,),
      in_specs=[
          pl.BlockSpec((block_rows, k), lambda i: (i, 0)),
          pl.BlockSpec((k, o), lambda i: (0, 0)),
          pl.BlockSpec((o,), lambda i: (0,)),
      ],
      out_specs=pl.BlockSpec((block_rows, o), lambda i: (i, 0)),
      out_shape=jax.ShapeDtypeStruct((m, o), jnp.float32),
  )(x, w, b)


def _sage_layer(hx, pp, degs, w_s, w_n, b, relu, block_rows=1264):
  m, k = hx.shape
  _, o = w_s.shape

  def body(h_ref, pp_ref, d_ref, ws_ref, wn_ref, b_ref, o_ref):
    recip = 1.0 / jnp.maximum(d_ref[0] + d_ref[1], 1.0)
    agg = (pp_ref[0] + pp_ref[1]) * recip
    acc = jnp.dot(h_ref[...], ws_ref[...], preferred_element_type=jnp.float32)
    acc = acc + jnp.dot(agg, wn_ref[...], preferred_element_type=jnp.float32)
    acc = acc + b_ref[...]
    if relu:
      acc = jnp.maximum(acc, 0.0)
    o_ref[...] = acc

  return pl.pallas_call(
      body,
      grid=(m // block_rows,),
      in_specs=[
          pl.BlockSpec((block_rows, k), lambda i: (i, 0)),
          pl.BlockSpec((NC, block_rows, k), lambda i: (0, i, 0)),
          pl.BlockSpec((NC, block_rows, 1), lambda i: (0, i, 0)),
          pl.BlockSpec((k, o), lambda i: (0, 0)),
          pl.BlockSpec((k, o), lambda i: (0, 0)),
          pl.BlockSpec((o,), lambda i: (0,)),
      ],
      out_specs=pl.BlockSpec((block_rows, o), lambda i: (i, 0)),
      out_shape=jax.ShapeDtypeStruct((m, o), jnp.float32),
  )(hx, pp, degs, w_s, w_n, b)


def kernel(d_features, m_features, edge_index, W_d, b_d, W_m, b_m,
           W_self1, W_neigh1, b1, W_self2, W_neigh2, b2):
  ei = edge_index.astype(jnp.int32)
  npad_s = NW * EPP_S - E
  pad_s = jnp.arange(npad_s, dtype=jnp.int32)
  src3 = jnp.concatenate([ei[0], pad_s % N]).reshape(NW, NCHUNK_S, CHUNK_S)
  dst3 = jnp.concatenate([ei[1], N + pad_s % (NP - N)]
                         ).reshape(NW, NCHUNK_S, CHUNK_S)
  npad_d = NW * EPP_D - E
  pad_d = jnp.arange(npad_d, dtype=jnp.int32)
  dst3d = jnp.concatenate([ei[1], N + pad_d % (NP - N)]
                          ).reshape(NW, NCHUNK_D, CHUNK_D)

  z2 = jnp.zeros((NP, EMB), jnp.float32)
  ones = jnp.ones((CHUNK_D, EMB), jnp.float32)

  degp = _deg_count(dst3d, z2, ones)
  degs = degp[:, :, 0:1]

  h_d = _linear(d_features, W_d, b_d, 1000)
  h_m = _linear(m_features, W_m, b_m, 1000)
  h = jnp.concatenate([h_d, h_m, jnp.zeros((NP - N, EMB), jnp.float32)],
                      axis=0)

  agg1 = _seg_sum(h, src3, dst3, z2)
  h1 = _sage_layer(h, agg1, degs, W_self1, W_neigh1, b1, True)

  agg2 = _seg_sum(h1, src3, dst3, z2)
  h2 = _sage_layer(h1, agg2, degs, W_self2, W_neigh2, b2, False)
  return h2[:N]
